# baseline (device time: 8319 ns/iter reference)
import jax
import jax.numpy as jnp
from jax import lax
from jax.experimental import pallas as pl
from jax.experimental.pallas import tpu as pltpu

N_DEV = 16
HALO = 3


def kernel(x, k):
    b, s, c = x.shape
    taps = k.shape[0]

    def body(x_ref, k_ref, out_ref, halo_ref, send_buf, send_sems, recv_sems):
        g = pl.program_id(0)
        my_i = lax.axis_index("i")
        left = lax.rem(my_i + N_DEV - 1, N_DEV)
        right = lax.rem(my_i + 1, N_DEV)
        is_sender = my_i < N_DEV - 1
        is_recver = my_i > 0

        credit_sem = pltpu.get_barrier_semaphore()

        @pl.when(g == 0)
        def _():
            @pl.when(is_recver)
            def _():
                pl.semaphore_signal(
                    credit_sem, inc=1,
                    device_id=(left,), device_id_type=pl.DeviceIdType.MESH,
                )

            @pl.when(is_sender)
            def _():
                pl.semaphore_wait(credit_sem, 1)

        send_buf[g, :, :] = x_ref[0, s - HALO:, :]
        rdma = pltpu.make_async_remote_copy(
            src_ref=send_buf.at[g],
            dst_ref=halo_ref.at[g],
            send_sem=send_sems.at[g],
            recv_sem=recv_sems.at[g],
            device_id=(right,),
            device_id_type=pl.DeviceIdType.MESH,
        )

        @pl.when(is_sender)
        def _():
            rdma.start()

        xv = x_ref[0, :, :].astype(jnp.bfloat16)
        kv = k_ref[:, :].astype(jnp.bfloat16)

        def silu_f32(a):
            return (a * jax.nn.sigmoid(a)).astype(jnp.float32)

        tail = xv[0:s - HALO, :] * kv[0, :][None, :]
        for t in range(1, taps):
            tail = tail + xv[t:t + s - HALO, :] * kv[t, :][None, :]
        out_ref[0, HALO:, :] = silu_f32(tail)

        @pl.when(is_recver)
        def _():
            rdma.wait_recv()

        halo = halo_ref[g, :, :].astype(jnp.bfloat16)
        halo = jnp.where(my_i == 0, jnp.zeros_like(halo), halo)
        hx = jnp.concatenate([halo, xv[:HALO, :]], axis=0)
        head = hx[0:HALO, :] * kv[0, :][None, :]
        for t in range(1, taps):
            head = head + hx[t:t + HALO, :] * kv[t, :][None, :]
        out_ref[0, :HALO, :] = silu_f32(head)

        @pl.when((g == b - 1) & is_sender)
        def _():
            for gg in range(b):
                rdma_gg = pltpu.make_async_remote_copy(
                    src_ref=send_buf.at[gg],
                    dst_ref=halo_ref.at[gg],
                    send_sem=send_sems.at[gg],
                    recv_sem=recv_sems.at[gg],
                    device_id=(right,),
                    device_id_type=pl.DeviceIdType.MESH,
                )
                rdma_gg.wait_send()

    return pl.pallas_call(
        body,
        grid=(b,),
        out_shape=jax.ShapeDtypeStruct((b, s, c), x.dtype),
        in_specs=[
            pl.BlockSpec((1, s, c), lambda g: (g, 0, 0), memory_space=pltpu.VMEM),
            pl.BlockSpec((taps, c), lambda g: (0, 0), memory_space=pltpu.VMEM),
        ],
        out_specs=pl.BlockSpec(
            (1, s, c), lambda g: (g, 0, 0), memory_space=pltpu.VMEM
        ),
        scratch_shapes=[
            pltpu.VMEM((b, HALO, c), x.dtype),
            pltpu.VMEM((b, HALO, c), x.dtype),
            pltpu.SemaphoreType.DMA((b,)),
            pltpu.SemaphoreType.DMA((b,)),
        ],
        compiler_params=pltpu.CompilerParams(collective_id=0),
    )(x, k)


# device time: 7186 ns/iter; 1.1577x vs baseline; 1.1577x over previous
import jax
import jax.numpy as jnp
from jax import lax
from jax.experimental import pallas as pl
from jax.experimental.pallas import tpu as pltpu

N_DEV = 16
HALO = 3
GB = 2


def kernel(x, k):
    b, s, c = x.shape
    taps = k.shape[0]
    n_steps = b // GB

    def body(x_ref, k_ref, out_ref, halo_ref, send_buf, send_sems, recv_sems):
        g = pl.program_id(0)
        my_i = lax.axis_index("i")
        left = lax.rem(my_i + N_DEV - 1, N_DEV)
        right = lax.rem(my_i + 1, N_DEV)
        is_sender = my_i < N_DEV - 1
        is_recver = my_i > 0

        credit_sem = pltpu.get_barrier_semaphore()

        @pl.when(g == 0)
        def _():
            @pl.when(is_recver)
            def _():
                pl.semaphore_signal(
                    credit_sem, inc=1,
                    device_id=(left,), device_id_type=pl.DeviceIdType.MESH,
                )

            @pl.when(is_sender)
            def _():
                pl.semaphore_wait(credit_sem, 1)

        send_buf[g, :, :, :] = x_ref[:, s - HALO:, :]
        rdma = pltpu.make_async_remote_copy(
            src_ref=send_buf.at[g],
            dst_ref=halo_ref.at[g],
            send_sem=send_sems.at[g],
            recv_sem=recv_sems.at[g],
            device_id=(right,),
            device_id_type=pl.DeviceIdType.MESH,
        )

        @pl.when(is_sender)
        def _():
            rdma.start()

        xv = x_ref[:, :, :].astype(jnp.bfloat16)
        kv = k_ref[:, :].astype(jnp.bfloat16)

        def silu_f32(a):
            return (a * jax.nn.sigmoid(a)).astype(jnp.float32)

        tail = xv[:, 0:s - HALO, :] * kv[0, :][None, None, :]
        for t in range(1, taps):
            tail = tail + xv[:, t:t + s - HALO, :] * kv[t, :][None, None, :]
        out_ref[:, HALO:, :] = silu_f32(tail)

        @pl.when(is_recver)
        def _():
            rdma.wait_recv()

        halo = halo_ref[g, :, :, :].astype(jnp.bfloat16)
        halo = jnp.where(my_i == 0, jnp.zeros_like(halo), halo)
        hx = jnp.concatenate([halo, xv[:, :HALO, :]], axis=1)
        head = hx[:, 0:HALO, :] * kv[0, :][None, None, :]
        for t in range(1, taps):
            head = head + hx[:, t:t + HALO, :] * kv[t, :][None, None, :]
        out_ref[:, :HALO, :] = silu_f32(head)

        @pl.when((g == n_steps - 1) & is_sender)
        def _():
            for gg in range(n_steps):
                rdma_gg = pltpu.make_async_remote_copy(
                    src_ref=send_buf.at[gg],
                    dst_ref=halo_ref.at[gg],
                    send_sem=send_sems.at[gg],
                    recv_sem=recv_sems.at[gg],
                    device_id=(right,),
                    device_id_type=pl.DeviceIdType.MESH,
                )
                rdma_gg.wait_send()

    return pl.pallas_call(
        body,
        grid=(n_steps,),
        out_shape=jax.ShapeDtypeStruct((b, s, c), x.dtype),
        in_specs=[
            pl.BlockSpec((GB, s, c), lambda g: (g, 0, 0), memory_space=pltpu.VMEM),
            pl.BlockSpec((taps, c), lambda g: (0, 0), memory_space=pltpu.VMEM),
        ],
        out_specs=pl.BlockSpec(
            (GB, s, c), lambda g: (g, 0, 0), memory_space=pltpu.VMEM
        ),
        scratch_shapes=[
            pltpu.VMEM((n_steps, GB, HALO, c), x.dtype),
            pltpu.VMEM((n_steps, GB, HALO, c), x.dtype),
            pltpu.SemaphoreType.DMA((n_steps,)),
            pltpu.SemaphoreType.DMA((n_steps,)),
        ],
        compiler_params=pltpu.CompilerParams(collective_id=0),
    )(x, k)


# device time: 6310 ns/iter; 1.3184x vs baseline; 1.1388x over previous
import jax
import jax.numpy as jnp
from jax import lax
from jax.experimental import pallas as pl
from jax.experimental.pallas import tpu as pltpu

N_DEV = 16
HALO = 3


def kernel(x, k):
    b, s, c = x.shape
    taps = k.shape[0]

    def body(x_ref, k_ref, out_ref, halo_ref, send_sem, recv_sem):
        my_i = lax.axis_index("i")
        left = lax.rem(my_i + N_DEV - 1, N_DEV)
        right = lax.rem(my_i + 1, N_DEV)

        credit_sem = pltpu.get_barrier_semaphore()

        @pl.when(my_i > 0)
        def _():
            pl.semaphore_signal(
                credit_sem, inc=1,
                device_id=(left,), device_id_type=pl.DeviceIdType.MESH,
            )

        rdma = pltpu.make_async_remote_copy(
            src_ref=x_ref.at[:, pl.ds(s - HALO, HALO), :],
            dst_ref=halo_ref,
            send_sem=send_sem,
            recv_sem=recv_sem,
            device_id=(right,),
            device_id_type=pl.DeviceIdType.MESH,
        )

        xv = x_ref[:, :, :].astype(jnp.bfloat16)
        kv = k_ref[:, :].astype(jnp.bfloat16)
        na = 128

        def conv_tail(lo, n):
            acc = xv[:, lo:lo + n, :] * kv[0, :][None, None, :]
            for t in range(1, taps):
                acc = acc + xv[:, lo + t:lo + t + n, :] * kv[t, :][None, None, :]
            return acc

        def silu_f32(a):
            return (a * jax.nn.sigmoid(a)).astype(jnp.float32)

        tail_a = conv_tail(0, na)
        out_ref[:, HALO:HALO + na, :] = silu_f32(tail_a)

        @pl.when(my_i < N_DEV - 1)
        def _():
            pl.semaphore_wait(credit_sem, 1)
            rdma.start()

        tail_b = conv_tail(na, s - HALO - na)
        out_ref[:, HALO + na:, :] = silu_f32(tail_b)

        @pl.when(my_i > 0)
        def _():
            rdma.wait_recv()

        halo = halo_ref[:, :, :].astype(jnp.bfloat16)
        halo = jnp.where(my_i == 0, jnp.zeros_like(halo), halo)
        hx = jnp.concatenate([halo, xv[:, :HALO, :]], axis=1)
        head = hx[:, 0:HALO, :] * kv[0, :][None, None, :]
        for t in range(1, taps):
            head = head + hx[:, t:t + HALO, :] * kv[t, :][None, None, :]
        out_ref[:, :HALO, :] = silu_f32(head)

        @pl.when(my_i < N_DEV - 1)
        def _():
            rdma.wait_send()

    return pl.pallas_call(
        body,
        out_shape=jax.ShapeDtypeStruct((b, s, c), x.dtype),
        in_specs=[
            pl.BlockSpec(memory_space=pltpu.VMEM),
            pl.BlockSpec(memory_space=pltpu.VMEM),
        ],
        out_specs=pl.BlockSpec(memory_space=pltpu.VMEM),
        scratch_shapes=[
            pltpu.VMEM((b, HALO, c), x.dtype),
            pltpu.SemaphoreType.DMA,
            pltpu.SemaphoreType.DMA,
        ],
        compiler_params=pltpu.CompilerParams(collective_id=0),
    )(x, k)
